# baseline (speedup 1.0000x reference)
import functools

import jax
import jax.numpy as jnp
from jax import lax
from jax.experimental import pallas as pl
from jax.experimental.pallas import tpu as pltpu
from jax.experimental.pallas import tpu_sc as plsc

_LANES = 16
_IDX_CHUNK = 128


def _make_sc_kernel(B, V, D, n_workers, num_cores):
    b_per_w = B // n_workers
    n_chunks = b_per_w // _IDX_CHUNK
    mesh = plsc.VectorSubcoreMesh(core_axis_name="c", subcore_axis_name="s")

    @functools.partial(
        pl.kernel,
        out_type=jax.ShapeDtypeStruct((B,), jnp.float32),
        mesh=mesh,
        scratch_types=[
            pltpu.VMEM((b_per_w,), jnp.int32),
            pltpu.VMEM((b_per_w,), jnp.int32),
            pltpu.VMEM((b_per_w, D), jnp.float32),
            pltpu.VMEM((b_per_w, D), jnp.float32),
            pltpu.VMEM((b_per_w,), jnp.float32),
            pltpu.SemaphoreType.DMA,
        ],
        compiler_params=pltpu.CompilerParams(
            needs_layout_passes=False, use_tc_tiling_on_sc=False),
    )
    def sc_kernel(center_hbm, context_hbm, in_hbm, out_hbm, score_hbm,
                  cidx, xidx, crows, xrows, scores, sem):
        wid = lax.axis_index("s") * num_cores + lax.axis_index("c")
        base = wid * b_per_w
        pltpu.sync_copy(center_hbm.at[pl.ds(base, b_per_w)], cidx)
        pltpu.sync_copy(context_hbm.at[pl.ds(base, b_per_w)], xidx)

        copies = []
        for j in range(n_chunks):
            sl = pl.ds(j * _IDX_CHUNK, _IDX_CHUNK)
            copies.append(pltpu.async_copy(in_hbm.at[cidx.at[sl]], crows.at[sl], sem))
            copies.append(pltpu.async_copy(out_hbm.at[xidx.at[sl]], xrows.at[sl], sem))
        for c in copies:
            c.wait()

        def chunk_body(i, carry):
            rows = i * _LANES + lax.iota(jnp.int32, _LANES)
            acc = jnp.zeros((_LANES,), jnp.float32)
            for d in range(D):
                col = jnp.full((_LANES,), d, jnp.int32)
                gc = plsc.load_gather(crows, [rows, col])
                gx = plsc.load_gather(xrows, [rows, col])
                acc = acc + gc * gx
            scores[pl.ds(i * _LANES, _LANES)] = acc
            return carry

        lax.fori_loop(0, b_per_w // _LANES, chunk_body, 0)
        pltpu.sync_copy(scores, score_hbm.at[pl.ds(base, b_per_w)])

    return sc_kernel


def kernel(center, context, in_table, out_table):
    B, = center.shape
    V, D = in_table.shape
    info = plsc.get_sparse_core_info()
    n_workers = info.num_cores * info.num_subcores
    sc_kernel = _make_sc_kernel(B, V, D, n_workers, info.num_cores)
    return sc_kernel(center, context, in_table, out_table)
